# baseline (speedup 1.0000x reference)
import functools

import jax
import jax.numpy as jnp
from jax import lax
from jax.experimental import pallas as pl
from jax.experimental.pallas import tpu as pltpu
from jax.experimental.pallas import tpu_sc as plsc

WIDTH = 32
RES = 128
SCALE = 1.5
N_PTS = 1048576
NVOX = RES * RES * RES

NC = 2
NS = 16
LANES = 16
NW = NC * NS
PER_W = N_PTS // NW
CHUNK = 128
NCHUNK = PER_W // CHUNK

BLK_CELLS = 8 * RES
NBLK = NVOX // BLK_CELLS
BLK_PER_W = NBLK // NW


def _mesh():
    return plsc.VectorSubcoreMesh(core_axis_name="c", subcore_axis_name="s",
                                  num_cores=NC, num_subcores=NS)


def _relayout_body(vg_hbm, tab_hbm, inv, outv, sem_a, sem_b):
    cid = lax.axis_index("c")
    sid = lax.axis_index("s")
    wid = sid * NC + cid
    lane16w = lax.iota(jnp.int32, LANES) * (WIDTH // 2)
    sems = (sem_a, sem_b)

    def issue_in(b, buf):
        blk = wid * BLK_PER_W + b
        d = blk >> 4
        hb = blk & 15
        pltpu.async_copy(vg_hbm.at[:, d, pl.ds(hb * 8, 8), :],
                         inv.at[buf], sems[buf])

    def wait_in(buf):
        pltpu.make_async_copy(vg_hbm.at[:, 0, pl.ds(0, 8), :],
                              inv.at[buf], sems[buf]).wait()

    def compute(b, buf):
        NWORD = WIDTH // 2

        @plsc.parallel_loop(0, BLK_CELLS // LANES, unroll=2)
        def g_body(g):
            h = g >> 3
            w0 = (g & 7) * LANES
            gbase = g * (LANES * NWORD)
            for c in range(NWORD):
                a = inv[buf, c, h, pl.ds(w0, LANES)]
                b2 = inv[buf, c + NWORD, h, pl.ds(w0, LANES)]
                w = plsc.bitcast(
                    plsc.pack(a, b2, format=plsc.PackFormat.INTERLEAVED),
                    jnp.int32)
                plsc.store_scatter(outv, [lane16w + (gbase + c)], w)

        blk = wid * BLK_PER_W + b
        pltpu.sync_copy(outv, tab_hbm.at[pl.ds(blk * (BLK_CELLS * NWORD),
                                               BLK_CELLS * NWORD)])

    PAIRS = BLK_PER_W // 2
    issue_in(0, 0)
    issue_in(1, 1)

    def pair_body(k, carry):
        a = 2 * k
        wait_in(0)
        compute(a, 0)

        @pl.when(k < PAIRS - 1)
        def _():
            issue_in(a + 2, 0)

        wait_in(1)
        compute(a + 1, 1)

        @pl.when(k < PAIRS - 1)
        def _():
            issue_in(a + 3, 1)

        return carry

    lax.fori_loop(0, PAIRS, pair_body, 0)


def _sample_body(xt_hbm, tab_hbm, out_hbm, xv, wqv, idxv, rowsv, outv,
                 sem_a, sem_b):
    cid = lax.axis_index("c")
    sid = lax.axis_index("s")
    wid = sid * NC + cid
    base = wid * PER_W
    lane = lax.iota(jnp.int32, LANES)
    lane16 = lane + LANES
    band_lo = lane >> 3
    sub_lo = lane & 7
    band_hi = lane16 >> 3
    sub_hi = lane16 & 7
    sems = (sem_a, sem_b)

    def dma_x(t, buf):
        pt = base + t * CHUNK
        pltpu.sync_copy(xt_hbm.at[:, pl.ds(pt, CHUNK)], xv.at[buf])

    def issue_gathers(buf):
        for c in range(8):
            pltpu.async_copy(tab_hbm.at[idxv.at[buf, c]],
                             rowsv.at[buf, c], sems[buf])

    def wait_gathers(buf):
        for c in range(8):
            pltpu.make_async_copy(tab_hbm.at[idxv.at[buf, c]],
                                  rowsv.at[buf, c], sems[buf]).wait()

    def phase1(t, buf):
        for j in range(CHUNK // LANES):
            sl = pl.ds(j * LANES, LANES)
            px = xv[buf, 0, sl]
            py = xv[buf, 1, sl]
            pz = xv[buf, 2, sl]
            m = ((jnp.abs(px) < SCALE) & (jnp.abs(py) < SCALE)
                 & (jnp.abs(pz) < SCALE))
            mf = jnp.where(m, 1.0, 0.0).astype(jnp.float32)
            gx = (jnp.clip(px / SCALE, -1.0, 1.0) + 1.0) * 0.5 * (RES - 1)
            gy = (jnp.clip(py / SCALE, -1.0, 1.0) + 1.0) * 0.5 * (RES - 1)
            gz = (jnp.clip(pz / SCALE, -1.0, 1.0) + 1.0) * 0.5 * (RES - 1)
            x0 = gx.astype(jnp.int32)
            y0 = gy.astype(jnp.int32)
            z0 = gz.astype(jnp.int32)
            wx = gx - x0.astype(jnp.float32)
            wy = gy - y0.astype(jnp.float32)
            wz = gz - z0.astype(jnp.float32)
            x1 = jnp.minimum(x0 + 1, RES - 1)
            y1 = jnp.minimum(y0 + 1, RES - 1)
            z1 = jnp.minimum(z0 + 1, RES - 1)
            zy00 = z0 * (RES * RES) + y0 * RES
            zy01 = z0 * (RES * RES) + y1 * RES
            zy10 = z1 * (RES * RES) + y0 * RES
            zy11 = z1 * (RES * RES) + y1 * RES
            idxv[buf, 0, sl] = zy00 + x0
            idxv[buf, 1, sl] = zy00 + x1
            idxv[buf, 2, sl] = zy01 + x0
            idxv[buf, 3, sl] = zy01 + x1
            idxv[buf, 4, sl] = zy10 + x0
            idxv[buf, 5, sl] = zy10 + x1
            idxv[buf, 6, sl] = zy11 + x0
            idxv[buf, 7, sl] = zy11 + x1
            q = (lane + j * LANES) * 4
            plsc.store_scatter(wqv.at[buf], [q], wx)
            plsc.store_scatter(wqv.at[buf], [q + 1], wy)
            plsc.store_scatter(wqv.at[buf], [q + 2], (1.0 - wz) * mf)
            plsc.store_scatter(wqv.at[buf], [q + 3], wz * mf)

    def combine(t, buf):
        pt = base + t * CHUNK

        @plsc.parallel_loop(0, CHUNK, unroll=4)
        def pt_body(i):
            iv = jnp.full((LANES,), i, jnp.int32)
            wvec = wqv[buf, pl.ds(i * 4, LANES)]
            wx = wvec[0]
            wy = wvec[1]
            w0 = wvec[2]
            w1 = wvec[3]
            rs = []
            for c in range(8):
                wrd = plsc.bitcast(rowsv[buf, c, i, :], jnp.bfloat16)
                rs.append(plsc.unpack(wrd, format=plsc.PackFormat.INTERLEAVED))
            for h in range(WIDTH // LANES):
                r0 = rs[0][h]
                r1 = rs[1][h]
                r2 = rs[2][h]
                r3 = rs[3][h]
                r4 = rs[4][h]
                r5 = rs[5][h]
                r6 = rs[6][h]
                r7 = rs[7][h]
                a0 = r0 + wx * (r1 - r0)
                a1 = r2 + wx * (r3 - r2)
                a2 = r4 + wx * (r5 - r4)
                a3 = r6 + wx * (r7 - r6)
                b0 = a0 + wy * (a1 - a0)
                b1 = a2 + wy * (a3 - a2)
                band = band_lo if h == 0 else band_hi
                sub = sub_lo if h == 0 else sub_hi
                plsc.store_scatter(outv, [band, sub, iv], b0 * w0 + b1 * w1)

        pltpu.sync_copy(outv, out_hbm.at[:, pt >> 7])

    PAIRS = NCHUNK // 2
    dma_x(0, 0)
    phase1(0, 0)
    issue_gathers(0)

    def pair_body(k, carry):
        a = 2 * k
        dma_x(a + 1, 1)
        phase1(a + 1, 1)
        wait_gathers(0)
        issue_gathers(1)
        combine(a, 0)

        @pl.when(k < PAIRS - 1)
        def _():
            dma_x(a + 2, 0)
            phase1(a + 2, 0)
            issue_gathers(0)

        wait_gathers(1)
        combine(a + 1, 1)
        return carry

    lax.fori_loop(0, PAIRS, pair_body, 0)


@jax.jit
def _run(xt, vg4):
    relayout = pl.kernel(
        _relayout_body,
        out_type=jax.ShapeDtypeStruct((NVOX * (WIDTH // 2),), jnp.int32),
        mesh=_mesh(),
        compiler_params=pltpu.CompilerParams(needs_layout_passes=False,
                                             use_tc_tiling_on_sc=True),
        scratch_types=[
            pltpu.VMEM((2, WIDTH, 8, RES), jnp.float32),
            pltpu.VMEM((BLK_CELLS * (WIDTH // 2),), jnp.int32),
            pltpu.SemaphoreType.DMA,
            pltpu.SemaphoreType.DMA,
        ],
    )
    table = relayout(vg4).reshape(NVOX, WIDTH // 2)

    sample = pl.kernel(
        _sample_body,
        out_type=jax.ShapeDtypeStruct((WIDTH // 8, N_PTS // CHUNK, 8, CHUNK),
                                      jnp.float32),
        mesh=_mesh(),
        compiler_params=pltpu.CompilerParams(needs_layout_passes=False,
                                             use_tc_tiling_on_sc=False),
        scratch_types=[
            pltpu.VMEM((2, 3, CHUNK), jnp.float32),
            pltpu.VMEM((2, CHUNK * 4 + LANES), jnp.float32),
            pltpu.VMEM((2, 8, CHUNK), jnp.int32),
            pltpu.VMEM((2, 8, CHUNK, WIDTH // 2), jnp.int32),
            pltpu.VMEM((WIDTH // 8, 8, CHUNK), jnp.float32),
            pltpu.SemaphoreType.DMA,
            pltpu.SemaphoreType.DMA,
        ],
    )
    return sample(xt, table)


def kernel(x, voxel_grid):
    vg4 = voxel_grid.reshape(WIDTH, RES, RES, RES)
    out4 = _run(x.T, vg4)
    out_t = out4.transpose(0, 2, 1, 3).reshape(WIDTH, N_PTS)
    return out_t.T


# speedup vs baseline: 1.0640x; 1.0640x over previous
import functools

import jax
import jax.numpy as jnp
from jax import lax
from jax.experimental import pallas as pl
from jax.experimental.pallas import tpu as pltpu
from jax.experimental.pallas import tpu_sc as plsc

WIDTH = 32
RES = 128
SCALE = 1.5
N_PTS = 1048576
NVOX = RES * RES * RES

NC = 2
NS = 16
LANES = 16
NW = NC * NS
PER_W = N_PTS // NW
CHUNK = 128
NCHUNK = PER_W // CHUNK

BLK_CELLS = 8 * RES
NBLK = NVOX // BLK_CELLS
BLK_PER_W = NBLK // NW


def _mesh():
    return plsc.VectorSubcoreMesh(core_axis_name="c", subcore_axis_name="s",
                                  num_cores=NC, num_subcores=NS)


def _relayout_body(vg_hbm, tab_hbm, inv, outv, sem_a, sem_b):
    cid = lax.axis_index("c")
    sid = lax.axis_index("s")
    wid = sid * NC + cid
    lane16w = lax.iota(jnp.int32, LANES) * (WIDTH // 2)
    sems = (sem_a, sem_b)

    def issue_in(b, buf):
        blk = wid * BLK_PER_W + b
        d = blk >> 4
        hb = blk & 15
        pltpu.async_copy(vg_hbm.at[:, d, pl.ds(hb * 8, 8), :],
                         inv.at[buf], sems[buf])

    def wait_in(buf):
        pltpu.make_async_copy(vg_hbm.at[:, 0, pl.ds(0, 8), :],
                              inv.at[buf], sems[buf]).wait()

    def compute(b, buf):
        NWORD = WIDTH // 2

        @plsc.parallel_loop(0, BLK_CELLS // LANES, unroll=2)
        def g_body(g):
            h = g >> 3
            w0 = (g & 7) * LANES
            gbase = g * (LANES * NWORD)
            for c in range(NWORD):
                a = inv[buf, c, h, pl.ds(w0, LANES)]
                b2 = inv[buf, c + NWORD, h, pl.ds(w0, LANES)]
                w = plsc.bitcast(
                    plsc.pack(a, b2, format=plsc.PackFormat.INTERLEAVED),
                    jnp.int32)
                plsc.store_scatter(outv, [lane16w + (gbase + c)], w)

        blk = wid * BLK_PER_W + b
        pltpu.sync_copy(outv, tab_hbm.at[pl.ds(blk * (BLK_CELLS * NWORD),
                                               BLK_CELLS * NWORD)])

    PAIRS = BLK_PER_W // 2
    issue_in(0, 0)
    issue_in(1, 1)

    def pair_body(k, carry):
        a = 2 * k
        wait_in(0)
        compute(a, 0)

        @pl.when(k < PAIRS - 1)
        def _():
            issue_in(a + 2, 0)

        wait_in(1)
        compute(a + 1, 1)

        @pl.when(k < PAIRS - 1)
        def _():
            issue_in(a + 3, 1)

        return carry

    lax.fori_loop(0, PAIRS, pair_body, 0)


def _sample_body(xt_hbm, tab_hbm, out_hbm, xv, wqv, idxv, rowsv, outv,
                 sem_a, sem_b):
    cid = lax.axis_index("c")
    sid = lax.axis_index("s")
    wid = sid * NC + cid
    base = wid * PER_W
    lane = lax.iota(jnp.int32, LANES)
    lane16 = lane + LANES
    band_lo = lane >> 3
    band_hi = lane16 >> 3
    inner_lo = (lane & 7) * CHUNK
    inner_hi = (lane16 & 7) * CHUNK
    sems = (sem_a, sem_b)

    def dma_x(t, buf):
        pt = base + t * CHUNK
        pltpu.sync_copy(xt_hbm.at[:, pl.ds(pt, CHUNK)], xv.at[buf])

    def issue_gathers(buf):
        for c in range(8):
            pltpu.async_copy(tab_hbm.at[idxv.at[buf, c]],
                             rowsv.at[buf, c], sems[buf])

    def wait_gathers(buf):
        for c in range(8):
            pltpu.make_async_copy(tab_hbm.at[idxv.at[buf, c]],
                                  rowsv.at[buf, c], sems[buf]).wait()

    def phase1(t, buf):
        for j in range(CHUNK // LANES):
            sl = pl.ds(j * LANES, LANES)
            px = xv[buf, 0, sl]
            py = xv[buf, 1, sl]
            pz = xv[buf, 2, sl]
            m = ((jnp.abs(px) < SCALE) & (jnp.abs(py) < SCALE)
                 & (jnp.abs(pz) < SCALE))
            mf = jnp.where(m, 1.0, 0.0).astype(jnp.float32)
            half_res = 0.5 * (RES - 1)
            inv_scale = 1.0 / SCALE
            gx = (jnp.clip(px * inv_scale, -1.0, 1.0) + 1.0) * half_res
            gy = (jnp.clip(py * inv_scale, -1.0, 1.0) + 1.0) * half_res
            gz = (jnp.clip(pz * inv_scale, -1.0, 1.0) + 1.0) * half_res
            x0 = gx.astype(jnp.int32)
            y0 = gy.astype(jnp.int32)
            z0 = gz.astype(jnp.int32)
            wx = gx - x0.astype(jnp.float32)
            wy = gy - y0.astype(jnp.float32)
            wz = gz - z0.astype(jnp.float32)
            x1 = jnp.minimum(x0 + 1, RES - 1)
            y1 = jnp.minimum(y0 + 1, RES - 1)
            z1 = jnp.minimum(z0 + 1, RES - 1)
            zy00 = z0 * (RES * RES) + y0 * RES
            zy01 = z0 * (RES * RES) + y1 * RES
            zy10 = z1 * (RES * RES) + y0 * RES
            zy11 = z1 * (RES * RES) + y1 * RES
            idxv[buf, 0, sl] = zy00 + x0
            idxv[buf, 1, sl] = zy00 + x1
            idxv[buf, 2, sl] = zy01 + x0
            idxv[buf, 3, sl] = zy01 + x1
            idxv[buf, 4, sl] = zy10 + x0
            idxv[buf, 5, sl] = zy10 + x1
            idxv[buf, 6, sl] = zy11 + x0
            idxv[buf, 7, sl] = zy11 + x1
            q = (lane + j * LANES) * 4
            plsc.store_scatter(wqv.at[buf], [q], wx)
            plsc.store_scatter(wqv.at[buf], [q + 1], wy)
            plsc.store_scatter(wqv.at[buf], [q + 2], (1.0 - wz) * mf)
            plsc.store_scatter(wqv.at[buf], [q + 3], wz * mf)

    def combine(t, buf):
        pt = base + t * CHUNK

        @plsc.parallel_loop(0, CHUNK, unroll=2)
        def pt_body(i):
            wvec = wqv[buf, pl.ds(i * 4, LANES)]
            wx = jnp.full((LANES,), wvec[0], jnp.float32)
            wy = jnp.full((LANES,), wvec[1], jnp.float32)
            w0 = jnp.full((LANES,), wvec[2], jnp.float32)
            w1 = jnp.full((LANES,), wvec[3], jnp.float32)
            idx_lo = inner_lo + i
            idx_hi = inner_hi + i
            rs = []
            for c in range(8):
                wrd = plsc.bitcast(rowsv[buf, c, i, :], jnp.bfloat16)
                rs.append(plsc.unpack(wrd, format=plsc.PackFormat.INTERLEAVED))
            for h in range(WIDTH // LANES):
                r0 = rs[0][h]
                r1 = rs[1][h]
                r2 = rs[2][h]
                r3 = rs[3][h]
                r4 = rs[4][h]
                r5 = rs[5][h]
                r6 = rs[6][h]
                r7 = rs[7][h]
                a0 = r0 + wx * (r1 - r0)
                a1 = r2 + wx * (r3 - r2)
                a2 = r4 + wx * (r5 - r4)
                a3 = r6 + wx * (r7 - r6)
                b0 = a0 + wy * (a1 - a0)
                b1 = a2 + wy * (a3 - a2)
                band = band_lo if h == 0 else band_hi
                idx = idx_lo if h == 0 else idx_hi
                plsc.store_scatter(outv, [band, idx], b0 * w0 + b1 * w1)

        pltpu.sync_copy(outv, out_hbm.at[:, pl.ds((pt >> 7) * (8 * CHUNK),
                                                  8 * CHUNK)])

    PAIRS = NCHUNK // 2
    dma_x(0, 0)
    phase1(0, 0)
    issue_gathers(0)

    def pair_body(k, carry):
        a = 2 * k
        dma_x(a + 1, 1)
        phase1(a + 1, 1)
        wait_gathers(0)
        issue_gathers(1)
        combine(a, 0)

        @pl.when(k < PAIRS - 1)
        def _():
            dma_x(a + 2, 0)
            phase1(a + 2, 0)
            issue_gathers(0)

        wait_gathers(1)
        combine(a + 1, 1)
        return carry

    lax.fori_loop(0, PAIRS, pair_body, 0)


@jax.jit
def _run(xt, vg4):
    relayout = pl.kernel(
        _relayout_body,
        out_type=jax.ShapeDtypeStruct((NVOX * (WIDTH // 2),), jnp.int32),
        mesh=_mesh(),
        compiler_params=pltpu.CompilerParams(needs_layout_passes=False,
                                             use_tc_tiling_on_sc=True),
        scratch_types=[
            pltpu.VMEM((2, WIDTH, 8, RES), jnp.float32),
            pltpu.VMEM((BLK_CELLS * (WIDTH // 2),), jnp.int32),
            pltpu.SemaphoreType.DMA,
            pltpu.SemaphoreType.DMA,
        ],
    )
    table = relayout(vg4).reshape(NVOX, WIDTH // 2)

    sample = pl.kernel(
        _sample_body,
        out_type=jax.ShapeDtypeStruct((WIDTH // 8, N_PTS * 8), jnp.float32),
        mesh=_mesh(),
        compiler_params=pltpu.CompilerParams(needs_layout_passes=False,
                                             use_tc_tiling_on_sc=False),
        scratch_types=[
            pltpu.VMEM((2, 3, CHUNK), jnp.float32),
            pltpu.VMEM((2, CHUNK * 4 + LANES), jnp.float32),
            pltpu.VMEM((2, 8, CHUNK), jnp.int32),
            pltpu.VMEM((2, 8, CHUNK, WIDTH // 2), jnp.int32),
            pltpu.VMEM((WIDTH // 8, 8 * CHUNK), jnp.float32),
            pltpu.SemaphoreType.DMA,
            pltpu.SemaphoreType.DMA,
        ],
    )
    return sample(xt, table)


def kernel(x, voxel_grid):
    vg4 = voxel_grid.reshape(WIDTH, RES, RES, RES)
    out4 = _run(x.T, vg4)
    out4 = out4.reshape(WIDTH // 8, N_PTS // CHUNK, 8, CHUNK)
    out_t = out4.transpose(0, 2, 1, 3).reshape(WIDTH, N_PTS)
    return out_t.T


# speedup vs baseline: 1.1140x; 1.0471x over previous
import functools

import jax
import jax.numpy as jnp
from jax import lax
from jax.experimental import pallas as pl
from jax.experimental.pallas import tpu as pltpu
from jax.experimental.pallas import tpu_sc as plsc

WIDTH = 32
RES = 128
SCALE = 1.5
N_PTS = 1048576
NVOX = RES * RES * RES

NC = 2
NS = 16
LANES = 16
NW = NC * NS
PER_W = N_PTS // NW
CHUNK = 128
NCHUNK = PER_W // CHUNK

BLK_CELLS = 8 * RES
NBLK = NVOX // BLK_CELLS
BLK_PER_W = NBLK // NW


def _mesh():
    return plsc.VectorSubcoreMesh(core_axis_name="c", subcore_axis_name="s",
                                  num_cores=NC, num_subcores=NS)


def _relayout_body(vg_hbm, tab_hbm, inv, outv, sem_a, sem_b):
    cid = lax.axis_index("c")
    sid = lax.axis_index("s")
    wid = sid * NC + cid
    lane16w = lax.iota(jnp.int32, LANES) * (WIDTH // 2)
    sems = (sem_a, sem_b)

    def issue_in(b, buf):
        blk = wid * BLK_PER_W + b
        d = blk >> 4
        hb = blk & 15
        pltpu.async_copy(vg_hbm.at[:, d, pl.ds(hb * 8, 8), :],
                         inv.at[buf], sems[buf])

    def wait_in(buf):
        pltpu.make_async_copy(vg_hbm.at[:, 0, pl.ds(0, 8), :],
                              inv.at[buf], sems[buf]).wait()

    def compute(b, buf):
        NWORD = WIDTH // 2

        @plsc.parallel_loop(0, BLK_CELLS // LANES, unroll=2)
        def g_body(g):
            h = g >> 3
            w0 = (g & 7) * LANES
            gbase = g * (LANES * NWORD)
            for c in range(NWORD):
                a = inv[buf, c, h, pl.ds(w0, LANES)]
                b2 = inv[buf, c + NWORD, h, pl.ds(w0, LANES)]
                w = plsc.bitcast(
                    plsc.pack(a, b2, format=plsc.PackFormat.INTERLEAVED),
                    jnp.int32)
                plsc.store_scatter(outv, [lane16w + (gbase + c)], w)

        blk = wid * BLK_PER_W + b
        pltpu.sync_copy(outv, tab_hbm.at[pl.ds(blk * (BLK_CELLS * NWORD),
                                               BLK_CELLS * NWORD)])

    PAIRS = BLK_PER_W // 2
    issue_in(0, 0)
    issue_in(1, 1)

    def pair_body(k, carry):
        a = 2 * k
        wait_in(0)
        compute(a, 0)

        @pl.when(k < PAIRS - 1)
        def _():
            issue_in(a + 2, 0)

        wait_in(1)
        compute(a + 1, 1)

        @pl.when(k < PAIRS - 1)
        def _():
            issue_in(a + 3, 1)

        return carry

    lax.fori_loop(0, PAIRS, pair_body, 0)


def _sample_body(xt_hbm, tab_hbm, out_hbm, xv, wqv, idxv, rowsv, outv,
                 sem_a, sem_b, sem_oa, sem_ob):
    cid = lax.axis_index("c")
    sid = lax.axis_index("s")
    wid = sid * NC + cid
    base = wid * PER_W
    lane = lax.iota(jnp.int32, LANES)
    lane16 = lane + LANES
    band_lo = lane >> 3
    band_hi = lane16 >> 3
    inner_lo = (lane & 7) * CHUNK
    inner_hi = (lane16 & 7) * CHUNK
    sems = (sem_a, sem_b)
    osems = (sem_oa, sem_ob)

    def dma_x(t, buf):
        pt = base + t * CHUNK
        pltpu.sync_copy(xt_hbm.at[:, pl.ds(pt, CHUNK)], xv.at[buf])

    def issue_gathers(buf):
        for c in range(8):
            pltpu.async_copy(tab_hbm.at[idxv.at[buf, c]],
                             rowsv.at[buf, c], sems[buf])

    def wait_gathers(buf):
        for c in range(8):
            pltpu.make_async_copy(tab_hbm.at[idxv.at[buf, c]],
                                  rowsv.at[buf, c], sems[buf]).wait()

    def phase1(t, buf):
        for j in range(CHUNK // LANES):
            sl = pl.ds(j * LANES, LANES)
            px = xv[buf, 0, sl]
            py = xv[buf, 1, sl]
            pz = xv[buf, 2, sl]
            m = ((jnp.abs(px) < SCALE) & (jnp.abs(py) < SCALE)
                 & (jnp.abs(pz) < SCALE))
            mf = jnp.where(m, 1.0, 0.0).astype(jnp.float32)
            half_res = 0.5 * (RES - 1)
            inv_scale = 1.0 / SCALE
            gx = (jnp.clip(px * inv_scale, -1.0, 1.0) + 1.0) * half_res
            gy = (jnp.clip(py * inv_scale, -1.0, 1.0) + 1.0) * half_res
            gz = (jnp.clip(pz * inv_scale, -1.0, 1.0) + 1.0) * half_res
            x0 = gx.astype(jnp.int32)
            y0 = gy.astype(jnp.int32)
            z0 = gz.astype(jnp.int32)
            wx = gx - x0.astype(jnp.float32)
            wy = gy - y0.astype(jnp.float32)
            wz = gz - z0.astype(jnp.float32)
            x1 = jnp.minimum(x0 + 1, RES - 1)
            y1 = jnp.minimum(y0 + 1, RES - 1)
            z1 = jnp.minimum(z0 + 1, RES - 1)
            zy00 = z0 * (RES * RES) + y0 * RES
            zy01 = z0 * (RES * RES) + y1 * RES
            zy10 = z1 * (RES * RES) + y0 * RES
            zy11 = z1 * (RES * RES) + y1 * RES
            idxv[buf, 0, sl] = zy00 + x0
            idxv[buf, 1, sl] = zy00 + x1
            idxv[buf, 2, sl] = zy01 + x0
            idxv[buf, 3, sl] = zy01 + x1
            idxv[buf, 4, sl] = zy10 + x0
            idxv[buf, 5, sl] = zy10 + x1
            idxv[buf, 6, sl] = zy11 + x0
            idxv[buf, 7, sl] = zy11 + x1
            q = (lane + j * LANES) * 4
            plsc.store_scatter(wqv.at[buf], [q], wx)
            plsc.store_scatter(wqv.at[buf], [q + 1], wy)
            plsc.store_scatter(wqv.at[buf], [q + 2], (1.0 - wz) * mf)
            plsc.store_scatter(wqv.at[buf], [q + 3], wz * mf)

    def combine(t, buf):
        pt = base + t * CHUNK

        @pl.when(t >= 2)
        def _():
            pltpu.make_async_copy(
                outv.at[buf],
                out_hbm.at[:, pl.ds(0, 8 * CHUNK)], osems[buf]).wait()

        @plsc.parallel_loop(0, CHUNK, unroll=2)
        def pt_body(i):
            wvec = wqv[buf, pl.ds(i * 4, LANES)]
            wx = jnp.full((LANES,), wvec[0], jnp.float32)
            wy = jnp.full((LANES,), wvec[1], jnp.float32)
            w0 = jnp.full((LANES,), wvec[2], jnp.float32)
            w1 = jnp.full((LANES,), wvec[3], jnp.float32)
            idx_lo = inner_lo + i
            idx_hi = inner_hi + i
            rs = []
            for c in range(8):
                wrd = plsc.bitcast(rowsv[buf, c, i, :], jnp.bfloat16)
                rs.append(plsc.unpack(wrd, format=plsc.PackFormat.INTERLEAVED))
            for h in range(WIDTH // LANES):
                r0 = rs[0][h]
                r1 = rs[1][h]
                r2 = rs[2][h]
                r3 = rs[3][h]
                r4 = rs[4][h]
                r5 = rs[5][h]
                r6 = rs[6][h]
                r7 = rs[7][h]
                a0 = r0 + wx * (r1 - r0)
                a1 = r2 + wx * (r3 - r2)
                a2 = r4 + wx * (r5 - r4)
                a3 = r6 + wx * (r7 - r6)
                b0 = a0 + wy * (a1 - a0)
                b1 = a2 + wy * (a3 - a2)
                band = band_lo if h == 0 else band_hi
                idx = idx_lo if h == 0 else idx_hi
                plsc.store_scatter(outv.at[buf], [band, idx],
                                   b0 * w0 + b1 * w1)

        pltpu.async_copy(outv.at[buf],
                         out_hbm.at[:, pl.ds((pt >> 7) * (8 * CHUNK),
                                             8 * CHUNK)], osems[buf])

    PAIRS = NCHUNK // 2
    dma_x(0, 0)
    phase1(0, 0)
    issue_gathers(0)

    def pair_body(k, carry):
        a = 2 * k
        dma_x(a + 1, 1)
        phase1(a + 1, 1)
        wait_gathers(0)
        issue_gathers(1)
        combine(a, 0)

        @pl.when(k < PAIRS - 1)
        def _():
            dma_x(a + 2, 0)
            phase1(a + 2, 0)
            issue_gathers(0)

        wait_gathers(1)
        combine(a + 1, 1)
        return carry

    lax.fori_loop(0, PAIRS, pair_body, 0)
    for buf in range(2):
        pltpu.make_async_copy(outv.at[buf],
                              out_hbm.at[:, pl.ds(0, 8 * CHUNK)],
                              osems[buf]).wait()


@jax.jit
def _run(xt, vg4):
    relayout = pl.kernel(
        _relayout_body,
        out_type=jax.ShapeDtypeStruct((NVOX * (WIDTH // 2),), jnp.int32),
        mesh=_mesh(),
        compiler_params=pltpu.CompilerParams(needs_layout_passes=False,
                                             use_tc_tiling_on_sc=True),
        scratch_types=[
            pltpu.VMEM((2, WIDTH, 8, RES), jnp.float32),
            pltpu.VMEM((BLK_CELLS * (WIDTH // 2),), jnp.int32),
            pltpu.SemaphoreType.DMA,
            pltpu.SemaphoreType.DMA,
        ],
    )
    table = relayout(vg4).reshape(NVOX, WIDTH // 2)

    sample = pl.kernel(
        _sample_body,
        out_type=jax.ShapeDtypeStruct((WIDTH // 8, N_PTS * 8), jnp.float32),
        mesh=_mesh(),
        compiler_params=pltpu.CompilerParams(needs_layout_passes=False,
                                             use_tc_tiling_on_sc=False),
        scratch_types=[
            pltpu.VMEM((2, 3, CHUNK), jnp.float32),
            pltpu.VMEM((2, CHUNK * 4 + LANES), jnp.float32),
            pltpu.VMEM((2, 8, CHUNK), jnp.int32),
            pltpu.VMEM((2, 8, CHUNK, WIDTH // 2), jnp.int32),
            pltpu.VMEM((2, WIDTH // 8, 8 * CHUNK), jnp.float32),
            pltpu.SemaphoreType.DMA,
            pltpu.SemaphoreType.DMA,
            pltpu.SemaphoreType.DMA,
            pltpu.SemaphoreType.DMA,
        ],
    )
    return sample(xt, table)


def kernel(x, voxel_grid):
    vg4 = voxel_grid.reshape(WIDTH, RES, RES, RES)
    out4 = _run(x.T, vg4)
    out4 = out4.reshape(WIDTH // 8, N_PTS // CHUNK, 8, CHUNK)
    out_t = out4.transpose(0, 2, 1, 3).reshape(WIDTH, N_PTS)
    return out_t.T


# speedup vs baseline: 1.2041x; 1.0808x over previous
import functools

import jax
import jax.numpy as jnp
from jax import lax
from jax.experimental import pallas as pl
from jax.experimental.pallas import tpu as pltpu
from jax.experimental.pallas import tpu_sc as plsc

WIDTH = 32
RES = 128
SCALE = 1.5
N_PTS = 1048576
NVOX = RES * RES * RES

NC = 2
NS = 16
LANES = 16
NW = NC * NS
PER_W = N_PTS // NW
CHUNK = 128
NCHUNK = PER_W // CHUNK

BLK_CELLS = 8 * RES
NBLK = NVOX // BLK_CELLS
BLK_PER_W = NBLK // NW


def _mesh():
    return plsc.VectorSubcoreMesh(core_axis_name="c", subcore_axis_name="s",
                                  num_cores=NC, num_subcores=NS)


def _relayout_body(vg_hbm, tab_hbm, inv, outv, sem_a, sem_b):
    cid = lax.axis_index("c")
    sid = lax.axis_index("s")
    wid = sid * NC + cid
    lane16w = lax.iota(jnp.int32, LANES) * (WIDTH // 2)
    sems = (sem_a, sem_b)

    def issue_in(b, buf):
        blk = wid * BLK_PER_W + b
        d = blk >> 4
        hb = blk & 15
        pltpu.async_copy(vg_hbm.at[:, d, pl.ds(hb * 8, 8), :],
                         inv.at[buf], sems[buf])

    def wait_in(buf):
        pltpu.make_async_copy(vg_hbm.at[:, 0, pl.ds(0, 8), :],
                              inv.at[buf], sems[buf]).wait()

    def compute(b, buf):
        NWORD = WIDTH // 2

        @plsc.parallel_loop(0, BLK_CELLS // LANES, unroll=2)
        def g_body(g):
            h = g >> 3
            w0 = (g & 7) * LANES
            gbase = g * (LANES * NWORD)
            for c in range(NWORD):
                a = inv[buf, c, h, pl.ds(w0, LANES)]
                b2 = inv[buf, c + NWORD, h, pl.ds(w0, LANES)]
                w = plsc.bitcast(
                    plsc.pack(a, b2, format=plsc.PackFormat.INTERLEAVED),
                    jnp.int32)
                plsc.store_scatter(outv, [lane16w + (gbase + c)], w)

        blk = wid * BLK_PER_W + b
        pltpu.sync_copy(outv, tab_hbm.at[pl.ds(blk * (BLK_CELLS * NWORD),
                                               BLK_CELLS * NWORD)])

    PAIRS = BLK_PER_W // 2
    issue_in(0, 0)
    issue_in(1, 1)

    def pair_body(k, carry):
        a = 2 * k
        wait_in(0)
        compute(a, 0)

        @pl.when(k < PAIRS - 1)
        def _():
            issue_in(a + 2, 0)

        wait_in(1)
        compute(a + 1, 1)

        @pl.when(k < PAIRS - 1)
        def _():
            issue_in(a + 3, 1)

        return carry

    lax.fori_loop(0, PAIRS, pair_body, 0)


def _sample_body(xt_hbm, tab_hbm, out_hbm, xv, wqv, idxv, rowsv, outv,
                 sem_a, sem_b, sem_oa, sem_ob):
    cid = lax.axis_index("c")
    sid = lax.axis_index("s")
    wid = sid * NC + cid
    base = wid * PER_W
    lane = lax.iota(jnp.int32, LANES)
    lane16 = lane + LANES
    band_lo = lane >> 3
    band_hi = lane16 >> 3
    inner_lo = (lane & 7) * CHUNK
    inner_hi = (lane16 & 7) * CHUNK
    sems = (sem_a, sem_b)
    osems = (sem_oa, sem_ob)

    def dma_x(t, buf):
        pt = base + t * CHUNK
        pltpu.sync_copy(xt_hbm.at[:, pl.ds(pt, CHUNK)], xv.at[buf])

    def issue_gathers(buf):
        for c in range(8):
            pltpu.async_copy(tab_hbm.at[idxv.at[buf, c]],
                             rowsv.at[buf, c], sems[buf])

    def wait_gathers(buf):
        for c in range(8):
            pltpu.make_async_copy(tab_hbm.at[idxv.at[buf, c]],
                                  rowsv.at[buf, c], sems[buf]).wait()

    def phase1(t, buf):
        for j in range(CHUNK // LANES):
            sl = pl.ds(j * LANES, LANES)
            px = xv[buf, 0, sl]
            py = xv[buf, 1, sl]
            pz = xv[buf, 2, sl]
            m = ((jnp.abs(px) < SCALE) & (jnp.abs(py) < SCALE)
                 & (jnp.abs(pz) < SCALE))
            mf = jnp.where(m, 1.0, 0.0).astype(jnp.float32)
            half_res = 0.5 * (RES - 1)
            inv_scale = 1.0 / SCALE
            gx = (jnp.clip(px * inv_scale, -1.0, 1.0) + 1.0) * half_res
            gy = (jnp.clip(py * inv_scale, -1.0, 1.0) + 1.0) * half_res
            gz = (jnp.clip(pz * inv_scale, -1.0, 1.0) + 1.0) * half_res
            x0 = gx.astype(jnp.int32)
            y0 = gy.astype(jnp.int32)
            z0 = gz.astype(jnp.int32)
            wx = gx - x0.astype(jnp.float32)
            wy = gy - y0.astype(jnp.float32)
            wz = gz - z0.astype(jnp.float32)
            x1 = jnp.minimum(x0 + 1, RES - 1)
            y1 = jnp.minimum(y0 + 1, RES - 1)
            z1 = jnp.minimum(z0 + 1, RES - 1)
            zy00 = z0 * (RES * RES) + y0 * RES
            zy01 = z0 * (RES * RES) + y1 * RES
            zy10 = z1 * (RES * RES) + y0 * RES
            zy11 = z1 * (RES * RES) + y1 * RES
            idxv[buf, 0, sl] = zy00 + x0
            idxv[buf, 1, sl] = zy00 + x1
            idxv[buf, 2, sl] = zy01 + x0
            idxv[buf, 3, sl] = zy01 + x1
            idxv[buf, 4, sl] = zy10 + x0
            idxv[buf, 5, sl] = zy10 + x1
            idxv[buf, 6, sl] = zy11 + x0
            idxv[buf, 7, sl] = zy11 + x1
            q = (lane + j * LANES) * 4
            plsc.store_scatter(wqv.at[buf], [q], wx)
            plsc.store_scatter(wqv.at[buf], [q + 1], wy)
            plsc.store_scatter(wqv.at[buf], [q + 2], (1.0 - wz) * mf)
            plsc.store_scatter(wqv.at[buf], [q + 3], wz * mf)

    def combine(t, buf):
        pt = base + t * CHUNK

        @pl.when(t >= 2)
        def _():
            pltpu.make_async_copy(
                outv.at[buf],
                out_hbm.at[:, pl.ds(0, 8 * CHUNK)], osems[buf]).wait()

        @plsc.parallel_loop(0, CHUNK, unroll=2)
        def pt_body(i):
            wvec = wqv[buf, pl.ds(i * 4, LANES)]
            wx16 = jnp.full((LANES,), wvec[0], jnp.float32)
            wxb = plsc.pack(wx16, wx16, format=plsc.PackFormat.INTERLEAVED)
            wy = jnp.full((LANES,), wvec[1], jnp.float32)
            w0 = jnp.full((LANES,), wvec[2], jnp.float32)
            w1 = jnp.full((LANES,), wvec[3], jnp.float32)
            idx_lo = inner_lo + i
            idx_hi = inner_hi + i
            ax = []
            for c in range(4):
                r0 = plsc.bitcast(rowsv[buf, 2 * c, i, :], jnp.bfloat16)
                r1 = plsc.bitcast(rowsv[buf, 2 * c + 1, i, :], jnp.bfloat16)
                a = r0 + wxb * (r1 - r0)
                ax.append(plsc.unpack(a, format=plsc.PackFormat.INTERLEAVED))
            for h in range(WIDTH // LANES):
                a0 = ax[0][h]
                a1 = ax[1][h]
                a2 = ax[2][h]
                a3 = ax[3][h]
                b0 = a0 + wy * (a1 - a0)
                b1 = a2 + wy * (a3 - a2)
                band = band_lo if h == 0 else band_hi
                idx = idx_lo if h == 0 else idx_hi
                plsc.store_scatter(outv.at[buf], [band, idx],
                                   b0 * w0 + b1 * w1)

        pltpu.async_copy(outv.at[buf],
                         out_hbm.at[:, pl.ds((pt >> 7) * (8 * CHUNK),
                                             8 * CHUNK)], osems[buf])

    PAIRS = NCHUNK // 2
    dma_x(0, 0)
    phase1(0, 0)
    issue_gathers(0)

    def pair_body(k, carry):
        a = 2 * k
        dma_x(a + 1, 1)
        phase1(a + 1, 1)
        wait_gathers(0)
        issue_gathers(1)
        combine(a, 0)

        @pl.when(k < PAIRS - 1)
        def _():
            dma_x(a + 2, 0)
            phase1(a + 2, 0)
            issue_gathers(0)

        wait_gathers(1)
        combine(a + 1, 1)
        return carry

    lax.fori_loop(0, PAIRS, pair_body, 0)
    for buf in range(2):
        pltpu.make_async_copy(outv.at[buf],
                              out_hbm.at[:, pl.ds(0, 8 * CHUNK)],
                              osems[buf]).wait()


@jax.jit
def _run(xt, vg4):
    relayout = pl.kernel(
        _relayout_body,
        out_type=jax.ShapeDtypeStruct((NVOX * (WIDTH // 2),), jnp.int32),
        mesh=_mesh(),
        compiler_params=pltpu.CompilerParams(needs_layout_passes=False,
                                             use_tc_tiling_on_sc=True),
        scratch_types=[
            pltpu.VMEM((2, WIDTH, 8, RES), jnp.float32),
            pltpu.VMEM((BLK_CELLS * (WIDTH // 2),), jnp.int32),
            pltpu.SemaphoreType.DMA,
            pltpu.SemaphoreType.DMA,
        ],
    )
    table = relayout(vg4).reshape(NVOX, WIDTH // 2)

    sample = pl.kernel(
        _sample_body,
        out_type=jax.ShapeDtypeStruct((WIDTH // 8, N_PTS * 8), jnp.float32),
        mesh=_mesh(),
        compiler_params=pltpu.CompilerParams(needs_layout_passes=False,
                                             use_tc_tiling_on_sc=False),
        scratch_types=[
            pltpu.VMEM((2, 3, CHUNK), jnp.float32),
            pltpu.VMEM((2, CHUNK * 4 + LANES), jnp.float32),
            pltpu.VMEM((2, 8, CHUNK), jnp.int32),
            pltpu.VMEM((2, 8, CHUNK, WIDTH // 2), jnp.int32),
            pltpu.VMEM((2, WIDTH // 8, 8 * CHUNK), jnp.float32),
            pltpu.SemaphoreType.DMA,
            pltpu.SemaphoreType.DMA,
            pltpu.SemaphoreType.DMA,
            pltpu.SemaphoreType.DMA,
        ],
    )
    return sample(xt, table)


def kernel(x, voxel_grid):
    vg4 = voxel_grid.reshape(WIDTH, RES, RES, RES)
    out4 = _run(x.T, vg4)
    out4 = out4.reshape(WIDTH // 8, N_PTS // CHUNK, 8, CHUNK)
    out_t = out4.transpose(0, 2, 1, 3).reshape(WIDTH, N_PTS)
    return out_t.T


# speedup vs baseline: 1.3808x; 1.1468x over previous
import functools

import jax
import jax.numpy as jnp
from jax import lax
from jax.experimental import pallas as pl
from jax.experimental.pallas import tpu as pltpu
from jax.experimental.pallas import tpu_sc as plsc

WIDTH = 32
RES = 128
SCALE = 1.5
N_PTS = 1048576
NVOX = RES * RES * RES

NC = 2
NS = 16
LANES = 16
NW = NC * NS
PER_W = N_PTS // NW
CHUNK = 128
NCHUNK = PER_W // CHUNK

BLK_CELLS = 8 * RES
NBLK = NVOX // BLK_CELLS
BLK_PER_W = NBLK // NW


def _mesh():
    return plsc.VectorSubcoreMesh(core_axis_name="c", subcore_axis_name="s",
                                  num_cores=NC, num_subcores=NS)


def _relayout_body(vg_hbm, tab_hbm, inv, outv, sem_a, sem_b):
    cid = lax.axis_index("c")
    sid = lax.axis_index("s")
    wid = sid * NC + cid
    lane16w = lax.iota(jnp.int32, LANES) * (WIDTH // 2)
    sems = (sem_a, sem_b)

    def issue_in(b, buf):
        blk = wid * BLK_PER_W + b
        d = blk >> 4
        hb = blk & 15
        pltpu.async_copy(vg_hbm.at[:, d, pl.ds(hb * 8, 8), :],
                         inv.at[buf], sems[buf])

    def wait_in(buf):
        pltpu.make_async_copy(vg_hbm.at[:, 0, pl.ds(0, 8), :],
                              inv.at[buf], sems[buf]).wait()

    def compute(b, buf):
        NWORD = WIDTH // 2

        @plsc.parallel_loop(0, BLK_CELLS // LANES, unroll=2)
        def g_body(g):
            h = g >> 3
            w0 = (g & 7) * LANES
            gbase = g * (LANES * NWORD)
            for c in range(NWORD):
                a = inv[buf, c, h, pl.ds(w0, LANES)]
                b2 = inv[buf, c + NWORD, h, pl.ds(w0, LANES)]
                w = plsc.bitcast(
                    plsc.pack(a, b2, format=plsc.PackFormat.INTERLEAVED),
                    jnp.int32)
                plsc.store_scatter(outv, [lane16w + (gbase + c)], w)

        blk = wid * BLK_PER_W + b
        pltpu.sync_copy(outv, tab_hbm.at[pl.ds(blk * (BLK_CELLS * NWORD),
                                               BLK_CELLS * NWORD)])

    PAIRS = BLK_PER_W // 2
    issue_in(0, 0)
    issue_in(1, 1)

    def pair_body(k, carry):
        a = 2 * k
        wait_in(0)
        compute(a, 0)

        @pl.when(k < PAIRS - 1)
        def _():
            issue_in(a + 2, 0)

        wait_in(1)
        compute(a + 1, 1)

        @pl.when(k < PAIRS - 1)
        def _():
            issue_in(a + 3, 1)

        return carry

    lax.fori_loop(0, PAIRS, pair_body, 0)


def _sample_body(xt_hbm, tab_hbm, out_hbm, xv, wqv, idxv, rowsv, outv,
                 sem_a, sem_b, sem_oa, sem_ob, sem_xa, sem_xb):
    cid = lax.axis_index("c")
    sid = lax.axis_index("s")
    wid = sid * NC + cid
    base = wid * PER_W
    lane = lax.iota(jnp.int32, LANES)
    lane16 = lane + LANES
    band_lo = lane >> 3
    band_hi = lane16 >> 3
    inner_lo = (lane & 7) * CHUNK
    inner_hi = (lane16 & 7) * CHUNK
    sems = (sem_a, sem_b)
    osems = (sem_oa, sem_ob)
    xsems = (sem_xa, sem_xb)

    def issue_x(t, buf):
        pt = base + t * CHUNK
        pltpu.async_copy(xt_hbm.at[:, pl.ds(pt, CHUNK)], xv.at[buf],
                         xsems[buf])

    def wait_x(buf):
        pltpu.make_async_copy(xt_hbm.at[:, pl.ds(0, CHUNK)], xv.at[buf],
                              xsems[buf]).wait()

    def issue_gathers(buf):
        for c in range(8):
            pltpu.async_copy(tab_hbm.at[idxv.at[buf, c]],
                             rowsv.at[buf, c], sems[buf])

    def wait_gathers(buf):
        for c in range(8):
            pltpu.make_async_copy(tab_hbm.at[idxv.at[buf, c]],
                                  rowsv.at[buf, c], sems[buf]).wait()

    def phase1(t, buf):
        for j in range(CHUNK // LANES):
            sl = pl.ds(j * LANES, LANES)
            px = xv[buf, 0, sl]
            py = xv[buf, 1, sl]
            pz = xv[buf, 2, sl]
            m = ((jnp.abs(px) < SCALE) & (jnp.abs(py) < SCALE)
                 & (jnp.abs(pz) < SCALE))
            mf = jnp.where(m, 1.0, 0.0).astype(jnp.float32)
            half_res = 0.5 * (RES - 1)
            inv_scale = 1.0 / SCALE
            gx = (jnp.clip(px * inv_scale, -1.0, 1.0) + 1.0) * half_res
            gy = (jnp.clip(py * inv_scale, -1.0, 1.0) + 1.0) * half_res
            gz = (jnp.clip(pz * inv_scale, -1.0, 1.0) + 1.0) * half_res
            x0 = gx.astype(jnp.int32)
            y0 = gy.astype(jnp.int32)
            z0 = gz.astype(jnp.int32)
            wx = gx - x0.astype(jnp.float32)
            wy = gy - y0.astype(jnp.float32)
            wz = gz - z0.astype(jnp.float32)
            x1 = jnp.minimum(x0 + 1, RES - 1)
            y1 = jnp.minimum(y0 + 1, RES - 1)
            z1 = jnp.minimum(z0 + 1, RES - 1)
            zy00 = z0 * (RES * RES) + y0 * RES
            zy01 = z0 * (RES * RES) + y1 * RES
            zy10 = z1 * (RES * RES) + y0 * RES
            zy11 = z1 * (RES * RES) + y1 * RES
            idxv[buf, 0, sl] = zy00 + x0
            idxv[buf, 1, sl] = zy00 + x1
            idxv[buf, 2, sl] = zy01 + x0
            idxv[buf, 3, sl] = zy01 + x1
            idxv[buf, 4, sl] = zy10 + x0
            idxv[buf, 5, sl] = zy10 + x1
            idxv[buf, 6, sl] = zy11 + x0
            idxv[buf, 7, sl] = zy11 + x1
            q = (lane + j * LANES) * 4
            plsc.store_scatter(wqv.at[buf], [q], wx)
            plsc.store_scatter(wqv.at[buf], [q + 1], wy)
            plsc.store_scatter(wqv.at[buf], [q + 2], (1.0 - wz) * mf)
            plsc.store_scatter(wqv.at[buf], [q + 3], wz * mf)

    def combine(t, buf):
        pt = base + t * CHUNK

        @pl.when(t >= 2)
        def _():
            pltpu.make_async_copy(
                outv.at[buf],
                out_hbm.at[:, pl.ds(0, 8 * CHUNK)], osems[buf]).wait()

        @plsc.parallel_loop(0, CHUNK, unroll=2)
        def pt_body(i):
            wvec = wqv[buf, pl.ds(i * 4, LANES)]
            wx16 = jnp.full((LANES,), wvec[0], jnp.float32)
            wxb = plsc.pack(wx16, wx16, format=plsc.PackFormat.INTERLEAVED)
            wy = jnp.full((LANES,), wvec[1], jnp.float32)
            w0 = jnp.full((LANES,), wvec[2], jnp.float32)
            w1 = jnp.full((LANES,), wvec[3], jnp.float32)
            idx_lo = inner_lo + i
            idx_hi = inner_hi + i
            ax = []
            for c in range(4):
                r0 = plsc.bitcast(rowsv[buf, 2 * c, i, :], jnp.bfloat16)
                r1 = plsc.bitcast(rowsv[buf, 2 * c + 1, i, :], jnp.bfloat16)
                a = r0 + wxb * (r1 - r0)
                ax.append(plsc.unpack(a, format=plsc.PackFormat.INTERLEAVED))
            for h in range(WIDTH // LANES):
                a0 = ax[0][h]
                a1 = ax[1][h]
                a2 = ax[2][h]
                a3 = ax[3][h]
                b0 = a0 + wy * (a1 - a0)
                b1 = a2 + wy * (a3 - a2)
                band = band_lo if h == 0 else band_hi
                idx = idx_lo if h == 0 else idx_hi
                plsc.store_scatter(outv.at[buf], [band, idx],
                                   b0 * w0 + b1 * w1)

        pltpu.async_copy(outv.at[buf],
                         out_hbm.at[:, pl.ds((pt >> 7) * (8 * CHUNK),
                                             8 * CHUNK)], osems[buf])

    PAIRS = NCHUNK // 2
    issue_x(0, 0)
    issue_x(1, 1)
    wait_x(0)
    phase1(0, 0)
    issue_gathers(0)

    def pair_body(k, carry):
        a = 2 * k

        @pl.when(k < PAIRS - 1)
        def _():
            issue_x(a + 2, 0)

        wait_x(1)
        phase1(a + 1, 1)

        @pl.when(k < PAIRS - 1)
        def _():
            issue_x(a + 3, 1)

        wait_gathers(0)
        issue_gathers(1)
        combine(a, 0)

        @pl.when(k < PAIRS - 1)
        def _():
            wait_x(0)
            phase1(a + 2, 0)
            issue_gathers(0)

        wait_gathers(1)
        combine(a + 1, 1)
        return carry

    lax.fori_loop(0, PAIRS, pair_body, 0)
    for buf in range(2):
        pltpu.make_async_copy(outv.at[buf],
                              out_hbm.at[:, pl.ds(0, 8 * CHUNK)],
                              osems[buf]).wait()


@jax.jit
def _run(xt, vg4):
    relayout = pl.kernel(
        _relayout_body,
        out_type=jax.ShapeDtypeStruct((NVOX * (WIDTH // 2),), jnp.int32),
        mesh=_mesh(),
        compiler_params=pltpu.CompilerParams(needs_layout_passes=False,
                                             use_tc_tiling_on_sc=True),
        scratch_types=[
            pltpu.VMEM((2, WIDTH, 8, RES), jnp.float32),
            pltpu.VMEM((BLK_CELLS * (WIDTH // 2),), jnp.int32),
            pltpu.SemaphoreType.DMA,
            pltpu.SemaphoreType.DMA,
        ],
    )
    table = relayout(vg4).reshape(NVOX, WIDTH // 2)

    sample = pl.kernel(
        _sample_body,
        out_type=jax.ShapeDtypeStruct((WIDTH // 8, N_PTS * 8), jnp.float32),
        mesh=_mesh(),
        compiler_params=pltpu.CompilerParams(needs_layout_passes=False,
                                             use_tc_tiling_on_sc=False),
        scratch_types=[
            pltpu.VMEM((2, 3, CHUNK), jnp.float32),
            pltpu.VMEM((2, CHUNK * 4 + LANES), jnp.float32),
            pltpu.VMEM((2, 8, CHUNK), jnp.int32),
            pltpu.VMEM((2, 8, CHUNK, WIDTH // 2), jnp.int32),
            pltpu.VMEM((2, WIDTH // 8, 8 * CHUNK), jnp.float32),
            pltpu.SemaphoreType.DMA,
            pltpu.SemaphoreType.DMA,
            pltpu.SemaphoreType.DMA,
            pltpu.SemaphoreType.DMA,
            pltpu.SemaphoreType.DMA,
            pltpu.SemaphoreType.DMA,
        ],
    )
    return sample(xt, table)


def kernel(x, voxel_grid):
    vg4 = voxel_grid.reshape(WIDTH, RES, RES, RES)
    out4 = _run(x.T, vg4)
    out4 = out4.reshape(WIDTH // 8, N_PTS // CHUNK, 8, CHUNK)
    out_t = out4.transpose(0, 2, 1, 3).reshape(WIDTH, N_PTS)
    return out_t.T
